# baseline (speedup 1.0000x reference)
import jax
import jax.numpy as jnp
from jax.experimental import pallas as pl
from jax.experimental.pallas import tpu as pltpu

_D = 128


def _norm_of(deg):
    return jnp.where(deg > 0, jax.lax.rsqrt(jnp.maximum(deg, 1.0)), 0.0)


def _row_block(n):
    if n % 10000 == 0:
        return 10000
    return n


def _edge_block(e):
    for eb in (1000, 500, 100, 50, 10, 1):
        if e % eb == 0:
            return eb
    return 1


def _deg_body(src_ref, dst_ref, dout_ref, din_ref):
    @pl.when(pl.program_id(0) == 0)
    def _():
        dout_ref[...] = jnp.zeros(dout_ref.shape, dout_ref.dtype)
        din_ref[...] = jnp.zeros(din_ref.shape, din_ref.dtype)

    def body(e, c):
        s = src_ref[0, 0, e]
        d = dst_ref[0, 0, e]
        dout_ref[pl.ds(s, 1), :] = dout_ref[pl.ds(s, 1), :] + 1.0
        din_ref[pl.ds(d, 1), :] = din_ref[pl.ds(d, 1), :] + 1.0
        return c

    jax.lax.fori_loop(0, src_ref.shape[2], body, 0)


def _degrees(src2d, dst2d, n_src, n_dst):
    nb = src2d.shape[0]
    eb = src2d.shape[2]
    return pl.pallas_call(
        _deg_body,
        grid=(nb,),
        in_specs=[
            pl.BlockSpec((1, 1, eb), lambda i: (i, 0, 0), memory_space=pltpu.SMEM),
            pl.BlockSpec((1, 1, eb), lambda i: (i, 0, 0), memory_space=pltpu.SMEM),
        ],
        out_specs=[
            pl.BlockSpec((n_src, _D), lambda i: (0, 0)),
            pl.BlockSpec((n_dst, _D), lambda i: (0, 0)),
        ],
        out_shape=[
            jax.ShapeDtypeStruct((n_src, _D), jnp.float32),
            jax.ShapeDtypeStruct((n_dst, _D), jnp.float32),
        ],
    )(src2d, dst2d)


def _mm_body(x_ref, deg_ref, w_ref, o_ref):
    xn = x_ref[...] * _norm_of(deg_ref[...])
    o_ref[...] = jnp.dot(xn, w_ref[...], preferred_element_type=jnp.float32)


def _mm(x, deg_out, w):
    n = x.shape[0]
    b = _row_block(n)
    return pl.pallas_call(
        _mm_body,
        grid=(n // b,),
        in_specs=[
            pl.BlockSpec((b, _D), lambda i: (i, 0)),
            pl.BlockSpec((b, _D), lambda i: (i, 0)),
            pl.BlockSpec((_D, _D), lambda i: (0, 0)),
        ],
        out_specs=pl.BlockSpec((b, _D), lambda i: (i, 0)),
        out_shape=jax.ShapeDtypeStruct((n, _D), jnp.float32),
    )(x, deg_out, w)


def _edge_body(src_ref, dst_ref, y_ref, acc_ref):
    @pl.when(pl.program_id(0) == 0)
    def _():
        acc_ref[...] = jnp.zeros(acc_ref.shape, acc_ref.dtype)

    def body(e, c):
        s = src_ref[0, 0, e]
        d = dst_ref[0, 0, e]
        acc_ref[pl.ds(d, 1), :] = acc_ref[pl.ds(d, 1), :] + y_ref[pl.ds(s, 1), :]
        return c

    jax.lax.fori_loop(0, src_ref.shape[2], body, 0)


def _edge_sum(src2d, dst2d, y, n_dst):
    nb = src2d.shape[0]
    eb = src2d.shape[2]
    n_src = y.shape[0]
    return pl.pallas_call(
        _edge_body,
        grid=(nb,),
        in_specs=[
            pl.BlockSpec((1, 1, eb), lambda i: (i, 0, 0), memory_space=pltpu.SMEM),
            pl.BlockSpec((1, 1, eb), lambda i: (i, 0, 0), memory_space=pltpu.SMEM),
            pl.BlockSpec((n_src, _D), lambda i: (0, 0)),
        ],
        out_specs=pl.BlockSpec((n_dst, _D), lambda i: (0, 0)),
        out_shape=jax.ShapeDtypeStruct((n_dst, _D), jnp.float32),
    )(src2d, dst2d, y)


def _post_body(acc_ref, deg_ref, b_ref, o_ref):
    o_ref[...] = acc_ref[...] * _norm_of(deg_ref[...]) + b_ref[...]


def _post(acc, deg_in, bias_row):
    n = acc.shape[0]
    b = _row_block(n)
    return pl.pallas_call(
        _post_body,
        grid=(n // b,),
        in_specs=[
            pl.BlockSpec((b, _D), lambda i: (i, 0)),
            pl.BlockSpec((b, _D), lambda i: (i, 0)),
            pl.BlockSpec((1, _D), lambda i: (0, 0)),
        ],
        out_specs=pl.BlockSpec((b, _D), lambda i: (i, 0)),
        out_shape=jax.ShapeDtypeStruct((n, _D), jnp.float32),
    )(acc, deg_in, bias_row)


def _combine_body(a_ref, b_ref, o_ref):
    o_ref[...] = jnp.maximum((a_ref[...] + b_ref[...]) * 0.5, 0.0)


def _combine(a, c):
    n = a.shape[0]
    b = _row_block(n)
    return pl.pallas_call(
        _combine_body,
        grid=(n // b,),
        in_specs=[
            pl.BlockSpec((b, _D), lambda i: (i, 0)),
            pl.BlockSpec((b, _D), lambda i: (i, 0)),
        ],
        out_specs=pl.BlockSpec((b, _D), lambda i: (i, 0)),
        out_shape=jax.ShapeDtypeStruct((n, _D), jnp.float32),
    )(a, c)


def _relu_body(x_ref, o_ref):
    o_ref[...] = jnp.maximum(x_ref[...], 0.0)


def _relu(x):
    n = x.shape[0]
    b = _row_block(n)
    return pl.pallas_call(
        _relu_body,
        grid=(n // b,),
        in_specs=[pl.BlockSpec((b, _D), lambda i: (i, 0))],
        out_specs=pl.BlockSpec((b, _D), lambda i: (i, 0)),
        out_shape=jax.ShapeDtypeStruct((n, _D), jnp.float32),
    )(x)


def _stats_body(x_ref, s1_ref, s2_ref):
    @pl.when(pl.program_id(0) == 0)
    def _():
        s1_ref[...] = jnp.zeros(s1_ref.shape, s1_ref.dtype)
        s2_ref[...] = jnp.zeros(s2_ref.shape, s2_ref.dtype)

    x = x_ref[...]
    s1_ref[...] += jnp.sum(x, axis=0, keepdims=True)
    s2_ref[...] += jnp.sum(x * x, axis=0, keepdims=True)


def _bn_apply_body(x_ref, s1_ref, s2_ref, g_ref, b_ref, o_ref, *, n):
    m = s1_ref[...] * (1.0 / n)
    v = s2_ref[...] * (1.0 / n) - m * m
    o_ref[...] = (x_ref[...] - m) * jax.lax.rsqrt(v + 1e-5) * g_ref[...] + b_ref[...]


def _batch_norm(x, g_row, b_row):
    import functools
    n = x.shape[0]
    b = _row_block(n)
    s1, s2 = pl.pallas_call(
        _stats_body,
        grid=(n // b,),
        in_specs=[pl.BlockSpec((b, _D), lambda i: (i, 0))],
        out_specs=[
            pl.BlockSpec((1, _D), lambda i: (0, 0)),
            pl.BlockSpec((1, _D), lambda i: (0, 0)),
        ],
        out_shape=[
            jax.ShapeDtypeStruct((1, _D), jnp.float32),
            jax.ShapeDtypeStruct((1, _D), jnp.float32),
        ],
    )(x)
    return pl.pallas_call(
        functools.partial(_bn_apply_body, n=float(n)),
        grid=(n // b,),
        in_specs=[
            pl.BlockSpec((b, _D), lambda i: (i, 0)),
            pl.BlockSpec((1, _D), lambda i: (0, 0)),
            pl.BlockSpec((1, _D), lambda i: (0, 0)),
            pl.BlockSpec((1, _D), lambda i: (0, 0)),
            pl.BlockSpec((1, _D), lambda i: (0, 0)),
        ],
        out_specs=pl.BlockSpec((b, _D), lambda i: (i, 0)),
        out_shape=jax.ShapeDtypeStruct((n, _D), jnp.float32),
    )(x, s1, s2, g_row, b_row)


def _conv(x, deg_out, deg_in, src2d, dst2d, n_dst, w, bias_row):
    y = _mm(x, deg_out, w)
    acc = _edge_sum(src2d, dst2d, y, n_dst)
    return _post(acc, deg_in, bias_row)


def kernel(seq_feat, label_feat, bt_src, bt_dst, inc_src, inc_dst, con_src,
           con_dst, W1_bt, b1_bt, W1_inc, b1_inc, W1_con, b1_con, W2_bt,
           b2_bt, W2_inc, b2_inc, W2_con, b2_con, bn1s_g, bn1s_b, bn1l_g,
           bn1l_b, bn2s_g, bn2s_b, bn2l_g, bn2l_b):
    n_seq = seq_feat.shape[0]
    n_lab = label_feat.shape[0]

    def e2d(e):
        eb = _edge_block(e.shape[0])
        return e.astype(jnp.int32).reshape(e.shape[0] // eb, 1, eb)

    bt_s, bt_d = e2d(bt_src), e2d(bt_dst)
    inc_s, inc_d = e2d(inc_src), e2d(inc_dst)
    con_s, con_d = e2d(con_src), e2d(con_dst)

    row = lambda v: v.reshape(1, _D)

    dout_bt, din_bt = _degrees(bt_s, bt_d, n_seq, n_lab)
    dout_inc, din_inc = _degrees(inc_s, inc_d, n_lab, n_seq)
    dout_con, din_con = _degrees(con_s, con_d, n_seq, n_seq)

    h_lab = _conv(seq_feat, dout_bt, din_bt, bt_s, bt_d, n_lab, W1_bt, row(b1_bt))
    h_seq_inc = _conv(label_feat, dout_inc, din_inc, inc_s, inc_d, n_seq, W1_inc, row(b1_inc))
    h_seq_con = _conv(seq_feat, dout_con, din_con, con_s, con_d, n_seq, W1_con, row(b1_con))
    t_seq = _combine(h_seq_inc, h_seq_con)
    t_lab = _relu(h_lab)
    h_seq = _batch_norm(t_seq, row(bn1s_g), row(bn1s_b))
    h_lab = _batch_norm(t_lab, row(bn1l_g), row(bn1l_b))

    h_lab2 = _conv(h_seq, dout_bt, din_bt, bt_s, bt_d, n_lab, W2_bt, row(b2_bt))
    h_seq_inc2 = _conv(h_lab, dout_inc, din_inc, inc_s, inc_d, n_seq, W2_inc, row(b2_inc))
    h_seq_con2 = _conv(h_seq, dout_con, din_con, con_s, con_d, n_seq, W2_con, row(b2_con))
    t_seq2 = _combine(h_seq_inc2, h_seq_con2)
    t_lab2 = _relu(h_lab2)
    h_seq2 = _batch_norm(t_seq2, row(bn2s_g), row(bn2s_b))
    h_lab2 = _batch_norm(t_lab2, row(bn2l_g), row(bn2l_b))

    return jnp.concatenate([h_seq2, h_lab2], axis=0)
